# baseline (speedup 1.0000x reference)
import jax
import jax.numpy as jnp
from jax.experimental import pallas as pl
from jax.experimental.pallas import tpu as pltpu

_ROWS = 3 * 1080
_COLS = 1920
_BM = 216


def _body(idx_ref, a_ref, b_ref, x_ref, o_ref):
    i = idx_ref[0]
    scale = jnp.exp(a_ref[i])
    shift = b_ref[i]
    o_ref[...] = x_ref[...] * scale + shift


def kernel(rendered_image, cur_index, exposure_a, exposure_b):
    x2d = rendered_image.reshape(_ROWS, _COLS)
    out = pl.pallas_call(
        _body,
        grid=(_ROWS // _BM,),
        in_specs=[
            pl.BlockSpec(memory_space=pltpu.SMEM),
            pl.BlockSpec(memory_space=pltpu.SMEM),
            pl.BlockSpec(memory_space=pltpu.SMEM),
            pl.BlockSpec((_BM, _COLS), lambda i: (i, 0)),
        ],
        out_specs=pl.BlockSpec((_BM, _COLS), lambda i: (i, 0)),
        out_shape=jax.ShapeDtypeStruct((_ROWS, _COLS), jnp.float32),
    )(cur_index, exposure_a.reshape(-1), exposure_b.reshape(-1), x2d)
    return out.reshape(rendered_image.shape)


# speedup vs baseline: 1.1699x; 1.1699x over previous
import jax
import jax.numpy as jnp
from jax.experimental import pallas as pl
from jax.experimental.pallas import tpu as pltpu

_ROWS = 3 * 1080
_COLS = 1920
_BM = 648


def _body(idx_ref, a_ref, b_ref, x_ref, o_ref):
    i = idx_ref[0]
    scale = jnp.exp(a_ref[i])
    shift = b_ref[i]
    o_ref[...] = x_ref[...] * scale + shift


def kernel(rendered_image, cur_index, exposure_a, exposure_b):
    x2d = rendered_image.reshape(_ROWS, _COLS)
    out = pl.pallas_call(
        _body,
        grid=(_ROWS // _BM,),
        in_specs=[
            pl.BlockSpec(memory_space=pltpu.SMEM),
            pl.BlockSpec(memory_space=pltpu.SMEM),
            pl.BlockSpec(memory_space=pltpu.SMEM),
            pl.BlockSpec((_BM, _COLS), lambda i: (i, 0)),
        ],
        out_specs=pl.BlockSpec((_BM, _COLS), lambda i: (i, 0)),
        out_shape=jax.ShapeDtypeStruct((_ROWS, _COLS), jnp.float32),
    )(cur_index, exposure_a.reshape(-1), exposure_b.reshape(-1), x2d)
    return out.reshape(rendered_image.shape)


# speedup vs baseline: 1.2619x; 1.0786x over previous
import jax
import jax.numpy as jnp
from jax.experimental import pallas as pl
from jax.experimental.pallas import tpu as pltpu

_ROWS = 3 * 1080
_COLS = 1920
_BM = 1080


def _body(idx_ref, a_ref, b_ref, x_ref, o_ref):
    i = idx_ref[0]
    scale = jnp.exp(a_ref[i])
    shift = b_ref[i]
    o_ref[...] = x_ref[...] * scale + shift


def kernel(rendered_image, cur_index, exposure_a, exposure_b):
    x2d = rendered_image.reshape(_ROWS, _COLS)
    out = pl.pallas_call(
        _body,
        grid=(_ROWS // _BM,),
        in_specs=[
            pl.BlockSpec(memory_space=pltpu.SMEM),
            pl.BlockSpec(memory_space=pltpu.SMEM),
            pl.BlockSpec(memory_space=pltpu.SMEM),
            pl.BlockSpec((_BM, _COLS), lambda i: (i, 0)),
        ],
        out_specs=pl.BlockSpec((_BM, _COLS), lambda i: (i, 0)),
        out_shape=jax.ShapeDtypeStruct((_ROWS, _COLS), jnp.float32),
    )(cur_index, exposure_a.reshape(-1), exposure_b.reshape(-1), x2d)
    return out.reshape(rendered_image.shape)


# speedup vs baseline: 1.3746x; 1.0893x over previous
import jax
import jax.numpy as jnp
from jax.experimental import pallas as pl
from jax.experimental.pallas import tpu as pltpu

_ROWS = 3 * 1080
_COLS = 1920
_BM = 1288


def _body(idx_ref, a_ref, b_ref, x_ref, o_ref):
    i = idx_ref[0]
    scale = jnp.exp(a_ref[i])
    shift = b_ref[i]
    o_ref[...] = x_ref[...] * scale + shift


def kernel(rendered_image, cur_index, exposure_a, exposure_b):
    x2d = rendered_image.reshape(_ROWS, _COLS)
    out = pl.pallas_call(
        _body,
        grid=(pl.cdiv(_ROWS, _BM),),
        in_specs=[
            pl.BlockSpec(memory_space=pltpu.SMEM),
            pl.BlockSpec(memory_space=pltpu.SMEM),
            pl.BlockSpec(memory_space=pltpu.SMEM),
            pl.BlockSpec((_BM, _COLS), lambda i: (i, 0)),
        ],
        out_specs=pl.BlockSpec((_BM, _COLS), lambda i: (i, 0)),
        out_shape=jax.ShapeDtypeStruct((_ROWS, _COLS), jnp.float32),
    )(cur_index, exposure_a.reshape(-1), exposure_b.reshape(-1), x2d)
    return out.reshape(rendered_image.shape)
